# baseline (speedup 1.0000x reference)
import functools

import jax
import jax.numpy as jnp
from jax import lax
from jax.experimental import pallas as pl
from jax.experimental.pallas import tpu as pltpu
from jax.experimental.pallas import tpu_sc as plsc

B, S = 2, 2048
DIM = 1024
INTER = 512
E = 8
K = 2
NG = 4
T = B * S
A_TOT = T * K
BLK = 256
G = A_TOT + E * BLK
NB = G // BLK
TB = 512
TBS = 256
NEG = -1e30

NC = 2
NS = 16
NWK = NC * NS
A_PER_W = A_TOT // NWK
CH = 64
NCH = A_PER_W // CH


def _router_body(lg_ref, rt_ref, cnt_ref):
    i = pl.program_id(0)

    @pl.when(i == 0)
    def _():
        cnt_ref[...] = jnp.zeros((8, 128), jnp.float32)

    lane = lax.broadcasted_iota(jnp.int32, (TB, 128), 1)
    lg = lg_ref[...]
    m = jnp.max(lg, axis=1, keepdims=True)
    ex = jnp.exp(lg - m)
    s = ex / jnp.sum(ex, axis=1, keepdims=True)

    row = lax.broadcasted_iota(jnp.int32, (128, 128), 0)
    col = lax.broadcasted_iota(jnp.int32, (128, 128), 1)
    gm = jnp.where((row < E) & (col < NG) & (row // 2 == col), 1.0, 0.0)
    gmt = jnp.where((col < E) & (row < NG) & (col // 2 == row), 1.0, 0.0)
    gs = jnp.dot(s, gm, preferred_element_type=jnp.float32,
                 precision=lax.Precision.HIGHEST)
    gsm = jnp.where(lane < NG, gs, NEG)

    g1 = jnp.max(gsm, axis=1, keepdims=True)
    i1 = jnp.min(jnp.where(gsm == g1, lane, 128), axis=1, keepdims=True)
    gsm2 = jnp.where(lane == i1, NEG, gsm)
    g2 = jnp.max(gsm2, axis=1, keepdims=True)
    i2 = jnp.min(jnp.where(gsm2 == g2, lane, 128), axis=1, keepdims=True)
    selg = jnp.where((lane == i1) | (lane == i2), 1.0, 0.0)
    sele = jnp.dot(selg, gmt, preferred_element_type=jnp.float32,
                   precision=lax.Precision.HIGHEST)

    sp = jnp.where((lane < E) & (sele > 0.5), s, NEG)
    v1 = jnp.max(sp, axis=1, keepdims=True)
    e1 = jnp.min(jnp.where(sp == v1, lane, 128), axis=1, keepdims=True)
    sp2 = jnp.where(lane == e1, NEG, sp)
    v2 = jnp.max(sp2, axis=1, keepdims=True)
    e2 = jnp.min(jnp.where(sp2 == v2, lane, 128), axis=1, keepdims=True)

    oh1 = jnp.where(lane == e1, 1.0, 0.0)
    oh2 = jnp.where(lane == e2, 1.0, 0.0)
    h = oh1 + oh2
    rr = lax.broadcasted_iota(jnp.int32, (TB, TB), 0)
    cc = lax.broadcasted_iota(jnp.int32, (TB, TB), 1)
    lf = jnp.where(cc < rr, 1.0, 0.0)
    p = jnp.dot(lf, h, preferred_element_type=jnp.float32)
    cb = cnt_ref[0:1, :]
    r1 = jnp.sum((p + cb) * oh1, axis=1, keepdims=True)
    r2 = jnp.sum((p + cb) * oh2, axis=1, keepdims=True)
    cnt_ref[0:1, :] = cb + jnp.sum(h, axis=0, keepdims=True)

    e1f = e1.astype(jnp.float32)
    e2f = e2.astype(jnp.float32)
    rt = jnp.where(lane == 0, e1f,
         jnp.where(lane == 1, e2f,
         jnp.where(lane == 2, r1,
         jnp.where(lane == 3, r2,
         jnp.where(lane == 4, v1,
         jnp.where(lane == 5, v2, 0.0))))))
    rt_ref[...] = rt


def _gmm_body(be_ref, x_ref, w1_ref, b1_ref, w2_ref, b2_ref, w3_ref, b3_ref,
              o_ref):
    a = x_ref[...]
    h1 = jnp.dot(a, w1_ref[0], preferred_element_type=jnp.float32) + b1_ref[0]
    h2 = jnp.dot(a, w2_ref[0], preferred_element_type=jnp.float32) + b2_ref[0]
    h = h1 * h2
    hs = h * jax.nn.sigmoid(h)
    o_ref[...] = jnp.dot(hs, w3_ref[0],
                         preferred_element_type=jnp.float32) + b3_ref[0]


def _combine_body(x_ref, ws1_ref, bs1_ref, ws2_ref, bs2_ref, ws3_ref, bs3_ref,
                  og_ref, rt_ref, y_ref):
    a = x_ref[...]
    h1 = jnp.dot(a, ws1_ref[...], preferred_element_type=jnp.float32) + bs1_ref[...]
    h2 = jnp.dot(a, ws2_ref[...], preferred_element_type=jnp.float32) + bs2_ref[...]
    h = h1 * h2
    hs = h * jax.nn.sigmoid(h)
    z = jnp.dot(hs, ws3_ref[...], preferred_element_type=jnp.float32) + bs3_ref[...]
    rt = rt_ref[...]
    v1 = rt[:, 4:5]
    v2 = rt[:, 5:6]
    og = og_ref[...]
    y_ref[...] = z + v1 * og[:, :DIM] + v2 * og[:, DIM:]


def _sc_dispatch_body(x_hbm, src_hbm, pos_hbm, xg_hbm, src_v, pos_v, rows_v,
                      sem1, sem2):
    wid = lax.axis_index("s") * NC + lax.axis_index("c")
    base = wid * NCH
    pltpu.sync_copy(src_hbm.at[pl.ds(base, NCH)], src_v)
    pltpu.sync_copy(pos_hbm.at[pl.ds(base, NCH)], pos_v)
    for j in range(NCH):
        pltpu.async_copy(x_hbm.at[src_v.at[j]], rows_v, sem1).wait()
        pltpu.async_copy(rows_v, xg_hbm.at[pos_v.at[j]], sem2).wait()


def _sc_gather_body(og_hbm, pos_hbm, out_hbm, pos_v, rows_v, sem):
    wid = lax.axis_index("s") * NC + lax.axis_index("c")
    base = wid * NCH
    abase = wid * A_PER_W
    pltpu.sync_copy(pos_hbm.at[pl.ds(base, NCH)], pos_v)
    for j in range(NCH):
        pltpu.async_copy(og_hbm.at[pos_v.at[j]], rows_v, sem).wait()
        pltpu.sync_copy(rows_v, out_hbm.at[pl.ds(abase + j * CH, CH)])


def _make_sc_kernels():
    mesh = plsc.VectorSubcoreMesh(core_axis_name="c", subcore_axis_name="s",
                                  num_cores=NC, num_subcores=NS)
    dispatch = pl.kernel(
        _sc_dispatch_body,
        out_type=jax.ShapeDtypeStruct((G, DIM), jnp.float32),
        mesh=mesh,
        scratch_types=[
            pltpu.VMEM((NCH, CH), jnp.int32),
            pltpu.VMEM((NCH, CH), jnp.int32),
            pltpu.VMEM((CH, DIM), jnp.float32),
            pltpu.SemaphoreType.DMA,
            pltpu.SemaphoreType.DMA,
        ],
    )
    gather = pl.kernel(
        _sc_gather_body,
        out_type=jax.ShapeDtypeStruct((A_TOT, DIM), jnp.float32),
        mesh=mesh,
        scratch_types=[
            pltpu.VMEM((NCH, CH), jnp.int32),
            pltpu.VMEM((CH, DIM), jnp.float32),
            pltpu.SemaphoreType.DMA,
        ],
    )
    return dispatch, gather


def kernel(x, Wr, br, We1, be1, We2, be2, We3, be3, Ws1, bs1, Ws2, bs2, Ws3,
           bs3):
    xf = x.reshape(T, DIM)
    logits = xf @ Wr + br
    lg_p = jnp.concatenate(
        [logits, jnp.full((T, 128 - E), NEG, jnp.float32)], axis=1)

    rt, cnts = pl.pallas_call(
        _router_body,
        grid=(T // TB,),
        in_specs=[
            pl.BlockSpec((TB, 128), lambda i: (i, 0)),
        ],
        out_specs=[
            pl.BlockSpec((TB, 128), lambda i: (i, 0)),
            pl.BlockSpec((8, 128), lambda i: (0, 0)),
        ],
        out_shape=[
            jax.ShapeDtypeStruct((T, 128), jnp.float32),
            jax.ShapeDtypeStruct((8, 128), jnp.float32),
        ],
    )(lg_p)

    counts = cnts[0, :E].astype(jnp.int32)
    nblk = (counts + BLK - 1) // BLK
    blk_cum = jnp.cumsum(nblk)
    starts = (blk_cum - nblk) * BLK
    ef = jnp.stack([rt[:, 0], rt[:, 1]], axis=1).reshape(-1).astype(jnp.int32)
    rf = jnp.stack([rt[:, 2], rt[:, 3]], axis=1).reshape(-1).astype(jnp.int32)
    pos = jnp.take(starts, ef) + rf
    blk_expert = jnp.minimum(
        jnp.searchsorted(blk_cum, jnp.arange(NB, dtype=jnp.int32),
                         side="right"),
        E - 1).astype(jnp.int32)

    pos2d = pos.reshape(NWK * NCH, CH)
    src2d = (jnp.arange(A_TOT, dtype=jnp.int32) // K).reshape(NWK * NCH, CH)

    sc_dispatch, sc_gather = _make_sc_kernels()
    x_g = sc_dispatch(xf, src2d, pos2d)

    og = pl.pallas_call(
        _gmm_body,
        grid_spec=pltpu.PrefetchScalarGridSpec(
            num_scalar_prefetch=1,
            grid=(NB,),
            in_specs=[
                pl.BlockSpec((BLK, DIM), lambda i, be: (i, 0)),
                pl.BlockSpec((1, DIM, INTER), lambda i, be: (be[i], 0, 0)),
                pl.BlockSpec((1, 1, INTER), lambda i, be: (be[i], 0, 0)),
                pl.BlockSpec((1, DIM, INTER), lambda i, be: (be[i], 0, 0)),
                pl.BlockSpec((1, 1, INTER), lambda i, be: (be[i], 0, 0)),
                pl.BlockSpec((1, INTER, DIM), lambda i, be: (be[i], 0, 0)),
                pl.BlockSpec((1, 1, DIM), lambda i, be: (be[i], 0, 0)),
            ],
            out_specs=pl.BlockSpec((BLK, DIM), lambda i, be: (i, 0)),
        ),
        out_shape=jax.ShapeDtypeStruct((G, DIM), jnp.float32),
    )(blk_expert, x_g, We1, be1.reshape(E, 1, INTER), We2,
      be2.reshape(E, 1, INTER), We3, be3.reshape(E, 1, DIM))

    og_tok = sc_gather(og, pos2d).reshape(T, 2 * DIM)

    y = pl.pallas_call(
        _combine_body,
        grid=(T // TBS,),
        in_specs=[
            pl.BlockSpec((TBS, DIM), lambda i: (i, 0)),
            pl.BlockSpec((DIM, 2 * INTER), lambda i: (0, 0)),
            pl.BlockSpec((1, 2 * INTER), lambda i: (0, 0)),
            pl.BlockSpec((DIM, 2 * INTER), lambda i: (0, 0)),
            pl.BlockSpec((1, 2 * INTER), lambda i: (0, 0)),
            pl.BlockSpec((2 * INTER, DIM), lambda i: (0, 0)),
            pl.BlockSpec((1, DIM), lambda i: (0, 0)),
            pl.BlockSpec((TBS, 2 * DIM), lambda i: (i, 0)),
            pl.BlockSpec((TBS, 128), lambda i: (i, 0)),
        ],
        out_specs=pl.BlockSpec((TBS, DIM), lambda i: (i, 0)),
        out_shape=jax.ShapeDtypeStruct((T, DIM), jnp.float32),
    )(xf, Ws1, bs1.reshape(1, -1), Ws2, bs2.reshape(1, -1), Ws3,
      bs3.reshape(1, -1), og_tok, rt)

    return y.reshape(x.shape)


# speedup vs baseline: 1.0303x; 1.0303x over previous
import jax
import jax.numpy as jnp
from jax import lax
from jax.experimental import pallas as pl
from jax.experimental.pallas import tpu as pltpu
from jax.experimental.pallas import tpu_sc as plsc

B, S = 2, 2048
DIM = 1024
INTER = 512
E = 8
K = 2
NG = 4
T = B * S
A_TOT = T * K
BLK = 256
G = A_TOT + E * BLK
NB = G // BLK
TB = 512
TBS = 512
NEG = -1e30

NC = 2
NS = 16
NWK = NC * NS
A_PER_W = A_TOT // NWK
CH = 64
NCH = A_PER_W // CH


def _router_body(lg_ref, rt_ref, cnt_ref):
    i = pl.program_id(0)

    @pl.when(i == 0)
    def _():
        cnt_ref[...] = jnp.zeros((8, 8), jnp.float32)

    lane = lax.broadcasted_iota(jnp.int32, (TB, E), 1)
    lg = lg_ref[...]
    m = jnp.max(lg, axis=1, keepdims=True)
    ex = jnp.exp(lg - m)
    s = ex / jnp.sum(ex, axis=1, keepdims=True)

    row = lax.broadcasted_iota(jnp.int32, (E, E), 0)
    col = lax.broadcasted_iota(jnp.int32, (E, E), 1)
    gm = jnp.where((col < NG) & (row // 2 == col), 1.0, 0.0)
    gmt = jnp.where((row < NG) & (col // 2 == row), 1.0, 0.0)
    gs = jnp.dot(s, gm, preferred_element_type=jnp.float32,
                 precision=lax.Precision.HIGHEST)
    gsm = jnp.where(lane < NG, gs, NEG)

    g1 = jnp.max(gsm, axis=1, keepdims=True)
    i1 = jnp.min(jnp.where(gsm == g1, lane, 128), axis=1, keepdims=True)
    gsm2 = jnp.where(lane == i1, NEG, gsm)
    g2 = jnp.max(gsm2, axis=1, keepdims=True)
    i2 = jnp.min(jnp.where(gsm2 == g2, lane, 128), axis=1, keepdims=True)
    selg = jnp.where((lane == i1) | (lane == i2), 1.0, 0.0)
    sele = jnp.dot(selg, gmt, preferred_element_type=jnp.float32,
                   precision=lax.Precision.HIGHEST)

    sp = jnp.where(sele > 0.5, s, NEG)
    v1 = jnp.max(sp, axis=1, keepdims=True)
    e1 = jnp.min(jnp.where(sp == v1, lane, 128), axis=1, keepdims=True)
    sp2 = jnp.where(lane == e1, NEG, sp)
    v2 = jnp.max(sp2, axis=1, keepdims=True)
    e2 = jnp.min(jnp.where(sp2 == v2, lane, 128), axis=1, keepdims=True)

    oh1 = jnp.where(lane == e1, 1.0, 0.0)
    oh2 = jnp.where(lane == e2, 1.0, 0.0)
    h = oh1 + oh2
    rr = lax.broadcasted_iota(jnp.int32, (TB, TB), 0)
    cc = lax.broadcasted_iota(jnp.int32, (TB, TB), 1)
    lf = jnp.where(cc < rr, 1.0, 0.0)
    p = jnp.dot(lf, h, preferred_element_type=jnp.float32)
    cb = cnt_ref[0:1, :]
    r1 = jnp.sum((p + cb) * oh1, axis=1, keepdims=True)
    r2 = jnp.sum((p + cb) * oh2, axis=1, keepdims=True)
    cnt_ref[0:1, :] = cb + jnp.sum(h, axis=0, keepdims=True)

    e1f = e1.astype(jnp.float32)
    e2f = e2.astype(jnp.float32)
    rt = jnp.where(lane == 0, e1f,
         jnp.where(lane == 1, e2f,
         jnp.where(lane == 2, r1,
         jnp.where(lane == 3, r2,
         jnp.where(lane == 4, v1,
         jnp.where(lane == 5, v2, 0.0))))))
    rt_ref[...] = rt


def _gmm_body(be_ref, x_ref, w1_ref, b1_ref, w2_ref, b2_ref, w3_ref, b3_ref,
              o_ref):
    a = x_ref[...]
    h1 = jnp.dot(a, w1_ref[0], preferred_element_type=jnp.float32) + b1_ref[0]
    h2 = jnp.dot(a, w2_ref[0], preferred_element_type=jnp.float32) + b2_ref[0]
    h = h1 * h2
    hs = h * jax.nn.sigmoid(h)
    o_ref[...] = jnp.dot(hs, w3_ref[0],
                         preferred_element_type=jnp.float32) + b3_ref[0]


def _shared_body(x_ref, ws1_ref, bs1_ref, ws2_ref, bs2_ref, ws3_ref, bs3_ref,
                 z_ref):
    a = x_ref[...]
    h1 = jnp.dot(a, ws1_ref[...], preferred_element_type=jnp.float32) + bs1_ref[...]
    h2 = jnp.dot(a, ws2_ref[...], preferred_element_type=jnp.float32) + bs2_ref[...]
    h = h1 * h2
    hs = h * jax.nn.sigmoid(h)
    z_ref[...] = jnp.dot(hs, ws3_ref[...],
                         preferred_element_type=jnp.float32) + bs3_ref[...]


def _combine_body(z_ref, og_ref, rt_ref, y_ref):
    rt = rt_ref[...]
    v1 = rt[:, 4:5]
    v2 = rt[:, 5:6]
    og = og_ref[...]
    y_ref[...] = z_ref[...] + v1 * og[:, :DIM] + v2 * og[:, DIM:]


def _sc_dispatch_body(x_hbm, src_hbm, pos_hbm, xg_hbm, src_v, pos_v, rows_v,
                      sem1, sem2):
    wid = lax.axis_index("s") * NC + lax.axis_index("c")
    base = wid * NCH
    pltpu.sync_copy(src_hbm.at[pl.ds(base, NCH)], src_v)
    pltpu.sync_copy(pos_hbm.at[pl.ds(base, NCH)], pos_v)
    for j in range(NCH):
        pltpu.async_copy(x_hbm.at[src_v.at[j]], rows_v, sem1).wait()
        pltpu.async_copy(rows_v, xg_hbm.at[pos_v.at[j]], sem2).wait()


def _sc_gather_body(og_hbm, pos_hbm, out_hbm, pos_v, rows_v, sem):
    wid = lax.axis_index("s") * NC + lax.axis_index("c")
    base = wid * NCH
    abase = wid * A_PER_W
    pltpu.sync_copy(pos_hbm.at[pl.ds(base, NCH)], pos_v)
    for j in range(NCH):
        pltpu.async_copy(og_hbm.at[pos_v.at[j]], rows_v, sem).wait()
        pltpu.sync_copy(rows_v, out_hbm.at[pl.ds(abase + j * CH, CH)])


def _make_sc_kernels():
    mesh = plsc.VectorSubcoreMesh(core_axis_name="c", subcore_axis_name="s",
                                  num_cores=NC, num_subcores=NS)
    dispatch = pl.kernel(
        _sc_dispatch_body,
        out_type=jax.ShapeDtypeStruct((G, DIM), jnp.float32),
        mesh=mesh,
        scratch_types=[
            pltpu.VMEM((NCH, CH), jnp.int32),
            pltpu.VMEM((NCH, CH), jnp.int32),
            pltpu.VMEM((CH, DIM), jnp.float32),
            pltpu.SemaphoreType.DMA,
            pltpu.SemaphoreType.DMA,
        ],
    )
    gather = pl.kernel(
        _sc_gather_body,
        out_type=jax.ShapeDtypeStruct((A_TOT, DIM), jnp.float32),
        mesh=mesh,
        scratch_types=[
            pltpu.VMEM((NCH, CH), jnp.int32),
            pltpu.VMEM((CH, DIM), jnp.float32),
            pltpu.SemaphoreType.DMA,
        ],
    )
    return dispatch, gather


def kernel(x, Wr, br, We1, be1, We2, be2, We3, be3, Ws1, bs1, Ws2, bs2, Ws3,
           bs3):
    xf = x.reshape(T, DIM)
    logits = xf @ Wr + br

    rt, cnts = pl.pallas_call(
        _router_body,
        grid=(T // TB,),
        in_specs=[
            pl.BlockSpec((TB, E), lambda i: (i, 0)),
        ],
        out_specs=[
            pl.BlockSpec((TB, E), lambda i: (i, 0)),
            pl.BlockSpec((8, 8), lambda i: (0, 0)),
        ],
        out_shape=[
            jax.ShapeDtypeStruct((T, E), jnp.float32),
            jax.ShapeDtypeStruct((8, 8), jnp.float32),
        ],
    )(logits)

    z = pl.pallas_call(
        _shared_body,
        grid=(T // TBS,),
        in_specs=[
            pl.BlockSpec((TBS, DIM), lambda i: (i, 0)),
            pl.BlockSpec((DIM, 2 * INTER), lambda i: (0, 0)),
            pl.BlockSpec((1, 2 * INTER), lambda i: (0, 0)),
            pl.BlockSpec((DIM, 2 * INTER), lambda i: (0, 0)),
            pl.BlockSpec((1, 2 * INTER), lambda i: (0, 0)),
            pl.BlockSpec((2 * INTER, DIM), lambda i: (0, 0)),
            pl.BlockSpec((1, DIM), lambda i: (0, 0)),
        ],
        out_specs=pl.BlockSpec((TBS, DIM), lambda i: (i, 0)),
        out_shape=jax.ShapeDtypeStruct((T, DIM), jnp.float32),
    )(xf, Ws1, bs1.reshape(1, -1), Ws2, bs2.reshape(1, -1), Ws3,
      bs3.reshape(1, -1))

    counts = cnts[0, :E].astype(jnp.int32)
    nblk = (counts + BLK - 1) // BLK
    blk_cum = jnp.cumsum(nblk)
    starts = (blk_cum - nblk) * BLK
    ef = jnp.stack([rt[:, 0], rt[:, 1]], axis=1).reshape(-1).astype(jnp.int32)
    rf = jnp.stack([rt[:, 2], rt[:, 3]], axis=1).reshape(-1).astype(jnp.int32)
    pos = jnp.take(starts, ef) + rf
    blk_expert = jnp.minimum(
        jnp.searchsorted(blk_cum, jnp.arange(NB, dtype=jnp.int32),
                         side="right"),
        E - 1).astype(jnp.int32)

    pos2d = pos.reshape(NWK * NCH, CH)
    src2d = (jnp.arange(A_TOT, dtype=jnp.int32) // K).reshape(NWK * NCH, CH)

    sc_dispatch, sc_gather = _make_sc_kernels()
    x_g = sc_dispatch(xf, src2d, pos2d)

    og = pl.pallas_call(
        _gmm_body,
        grid_spec=pltpu.PrefetchScalarGridSpec(
            num_scalar_prefetch=1,
            grid=(NB,),
            in_specs=[
                pl.BlockSpec((BLK, DIM), lambda i, be: (i, 0)),
                pl.BlockSpec((1, DIM, INTER), lambda i, be: (be[i], 0, 0)),
                pl.BlockSpec((1, 1, INTER), lambda i, be: (be[i], 0, 0)),
                pl.BlockSpec((1, DIM, INTER), lambda i, be: (be[i], 0, 0)),
                pl.BlockSpec((1, 1, INTER), lambda i, be: (be[i], 0, 0)),
                pl.BlockSpec((1, INTER, DIM), lambda i, be: (be[i], 0, 0)),
                pl.BlockSpec((1, 1, DIM), lambda i, be: (be[i], 0, 0)),
            ],
            out_specs=pl.BlockSpec((BLK, DIM), lambda i, be: (i, 0)),
        ),
        out_shape=jax.ShapeDtypeStruct((G, DIM), jnp.float32),
    )(blk_expert, x_g, We1, be1.reshape(E, 1, INTER), We2,
      be2.reshape(E, 1, INTER), We3, be3.reshape(E, 1, DIM))

    og_tok = sc_gather(og, pos2d).reshape(T, 2 * DIM)

    y = pl.pallas_call(
        _combine_body,
        grid=(T // TBS,),
        in_specs=[
            pl.BlockSpec((TBS, DIM), lambda i: (i, 0)),
            pl.BlockSpec((TBS, 2 * DIM), lambda i: (i, 0)),
            pl.BlockSpec((TBS, E), lambda i: (i, 0)),
        ],
        out_specs=pl.BlockSpec((TBS, DIM), lambda i: (i, 0)),
        out_shape=jax.ShapeDtypeStruct((T, DIM), jnp.float32),
    )(z, og_tok, rt)

    return y.reshape(x.shape)
